# baseline (speedup 1.0000x reference)
import jax
import jax.numpy as jnp
from jax import lax
from jax.experimental import pallas as pl
from jax.experimental.pallas import tpu as pltpu
from jax.experimental.pallas import tpu_sc as plsc

_N = 20000
_S = 16
_CH = 1280
_PADN = _S * _CH
_V = _CH // 16
_SCORE_THRESH = 0.2
_NMS_THRESH = 0.5
_BIG = float(2.0 ** 30)
_DNUMS = lax.GatherDimensionNumbers(
    offset_dims=(), collapsed_slice_dims=(0,), start_index_map=(0,))


def _permute(v, perm):
    return lax.gather(v, perm[:, None], _DNUMS, slice_sizes=(1,),
                      mode=lax.GatherScatterMode.PROMISE_IN_BOUNDS)


def _bfly_max(lane, best, idx):
    for d in (8, 4, 2, 1):
        perm = lane ^ d
        ob = _permute(best, perm)
        oi = _permute(idx, perm)
        better = ob > best
        tie = ob == best
        idx = jnp.where(better, oi,
                        jnp.where(tie, jnp.minimum(idx, oi), idx))
        best = jnp.maximum(best, ob)
    return best, idx


def _sc_body(x1h, y1h, x2h, y2h, sh, labh, outh,
             rx1, ry1, rx2, ry2, sv, lv,
             ox1, oy1, ox2, oy2, ar, esh, eso,
             stage, red, pdv, outv, shared, shared_pd):
    cid = lax.axis_index("c")
    sid = lax.axis_index("s")
    lane = lax.iota(jnp.int32, 16)

    @pl.when(cid == 0)
    def _main():
        base = sid * _CH
        pltpu.sync_copy(x1h.at[pl.ds(base, _CH)], rx1.at[pl.ds(0, _CH)])
        pltpu.sync_copy(y1h.at[pl.ds(base, _CH)], ry1.at[pl.ds(0, _CH)])
        pltpu.sync_copy(x2h.at[pl.ds(base, _CH)], rx2.at[pl.ds(0, _CH)])
        pltpu.sync_copy(y2h.at[pl.ds(base, _CH)], ry2.at[pl.ds(0, _CH)])
        pltpu.sync_copy(sh.at[pl.ds(base, _CH)], sv.at[pl.ds(0, _CH)])
        pltpu.sync_copy(labh.at[pl.ds(base, _CH)], lv.at[pl.ds(0, _CH)])

        def bmax_step(j, mv):
            sl = pl.ds(j * 16, 16)
            c = jnp.maximum(jnp.maximum(rx1[sl], ry1[sl]),
                            jnp.maximum(rx2[sl], ry2[sl]))
            c = jnp.where(sv[sl] >= _SCORE_THRESH, c, -jnp.inf)
            return jnp.maximum(mv, c)

        mv = lax.fori_loop(0, _V, bmax_step,
                           jnp.full((16,), -jnp.inf, jnp.float32))
        lbv, _ = _bfly_max(lane, mv, lane.astype(jnp.float32))
        stage[...] = jnp.where(lane == 0, lbv, 0.0)
        pltpu.sync_copy(stage, shared.at[pl.ds(sid * 16, 16)])
        plsc.subcore_barrier()
        pltpu.sync_copy(shared, red)

        def bred_step(j, m):
            rv = red[pl.ds(j * 16, 16)]
            return jnp.maximum(m, rv[0])

        bmax = lax.fori_loop(0, _S, bred_step, jnp.float32(-jnp.inf))
        plsc.subcore_barrier()

        def init_step(j, _):
            sl = pl.ds(j * 16, 16)
            labc = lv[sl]
            sc = sv[sl]
            off = labc * (bmax + 1.0)
            a1 = rx1[sl] + off
            b1 = ry1[sl] + off
            a2 = rx2[sl] + off
            b2 = ry2[sl] + off
            ox1[sl] = a1
            oy1[sl] = b1
            ox2[sl] = a2
            oy2[sl] = b2
            ar[sl] = (a2 - a1) * (b2 - b1)
            valid = sc >= _SCORE_THRESH
            esh[sl] = jnp.where(valid & (labc == 0.0), sc, -1.0)
            eso[sl] = jnp.where(valid & (labc != 0.0), sc, -1.0)
            return 0

        lax.fori_loop(0, _V, init_step, 0)

        def zero_step(j, _):
            outv[pl.ds(j * 16, 16)] = jnp.zeros((16,), jnp.float32)
            return 0

        lax.fori_loop(0, 32, zero_step, 0)

        def pick(r, es):
            def amax_step(j, bi):
                best, idx = bi
                v = es[pl.ds(j * 16, 16)]
                gt = v > best
                gidx = (base + j * 16 + lane).astype(jnp.float32)
                return (jnp.where(gt, v, best), jnp.where(gt, gidx, idx))

            best, idx = lax.fori_loop(
                0, _V, amax_step,
                (jnp.full((16,), -1.0, jnp.float32),
                 jnp.full((16,), _BIG, jnp.float32)))
            mlocv, ilocv = _bfly_max(lane, best, idx)
            stage[...] = jnp.where(lane == 0, mlocv,
                                   jnp.where(lane == 1, ilocv, 0.0))
            pltpu.sync_copy(stage, shared.at[pl.ds(sid * 16, 16)])
            plsc.subcore_barrier()
            pltpu.sync_copy(shared, red)

            def wred_step(j, wmi):
                wm, wi = wmi
                rv = red[pl.ds(j * 16, 16)]
                mj = rv[0]
                ij = rv[1]
                better = mj > wm
                tie = mj == wm
                wi = jnp.where(better, ij,
                               jnp.where(tie, jnp.minimum(wi, ij), wi))
                return jnp.maximum(wm, mj), wi

            wm, wif = lax.fori_loop(0, _S, wred_step,
                                    (jnp.float32(-2.0), jnp.float32(_BIG)))
            wi = wif.astype(jnp.int32)
            ok = wm >= 0.0
            owner = lax.div(wi, jnp.int32(_CH))

            @pl.when(ok & (owner == sid))
            def _owner():
                li = wi - base
                sl = pl.ds(li, 16)
                pd = jnp.where(lane == 0, ox1[sl][0],
                     jnp.where(lane == 1, oy1[sl][0],
                     jnp.where(lane == 2, ox2[sl][0],
                     jnp.where(lane == 3, oy2[sl][0],
                     jnp.where(lane == 4, ar[sl][0],
                     jnp.where(lane == 5, rx1[sl][0],
                     jnp.where(lane == 6, ry1[sl][0],
                     jnp.where(lane == 7, rx2[sl][0],
                     jnp.where(lane == 8, ry2[sl][0],
                     jnp.where(lane == 9, sv[sl][0],
                               0.0))))))))))
                stage[...] = pd
                pltpu.sync_copy(stage, shared_pd)

            plsc.subcore_barrier()
            pltpu.sync_copy(shared_pd, pdv)
            pdvec = pdv[...]
            pox1 = pdvec[0]
            poy1 = pdvec[1]
            pox2 = pdvec[2]
            poy2 = pdvec[3]
            par = pdvec[4]

            @pl.when(ok)
            def _sweep():
                def sw_step(j, _):
                    sl = pl.ds(j * 16, 16)
                    xx1 = jnp.maximum(pox1, ox1[sl])
                    yy1 = jnp.maximum(poy1, oy1[sl])
                    xx2 = jnp.minimum(pox2, ox2[sl])
                    yy2 = jnp.minimum(poy2, oy2[sl])
                    w = jnp.maximum(0.0, xx2 - xx1)
                    h = jnp.maximum(0.0, yy2 - yy1)
                    inter = w * h
                    iou = inter / (par + ar[sl] - inter + 1e-9)
                    es[sl] = jnp.where(iou > _NMS_THRESH, -1.0, es[sl])
                    return 0

                lax.fori_loop(0, _V, sw_step, 0)

            @pl.when(ok & (sid == 0))
            def _emit():
                row = jnp.where(lane == 0, pdvec[5],
                      jnp.where(lane == 1, pdvec[6],
                      jnp.where(lane == 2, pdvec[7],
                      jnp.where(lane == 3, pdvec[8],
                      jnp.where(lane == 4, pdvec[9],
                                0.0)))))
                outv[pl.ds(r * 16, 16)] = row

            return r + ok.astype(jnp.int32)

        r = lax.fori_loop(0, 15, lambda i, r: pick(r, esh), jnp.int32(0))
        lax.fori_loop(0, 15, lambda i, r: pick(r, eso), r)

        @pl.when(sid == 0)
        def _out():
            pltpu.sync_copy(outv, outh)


def _sc_call(x1, y1, x2, y2, s, lab):
    mesh = plsc.VectorSubcoreMesh(core_axis_name="c", subcore_axis_name="s")
    f = pl.kernel(
        _sc_body,
        out_type=jax.ShapeDtypeStruct((512,), jnp.float32),
        mesh=mesh,
        scratch_types=[
            pltpu.VMEM((_CH + 16,), jnp.float32) for _ in range(13)
        ] + [
            pltpu.VMEM((16,), jnp.float32),
            pltpu.VMEM((_S * 16,), jnp.float32),
            pltpu.VMEM((16,), jnp.float32),
            pltpu.VMEM((512,), jnp.float32),
            pltpu.VMEM_SHARED((_S * 16,), jnp.float32),
            pltpu.VMEM_SHARED((16,), jnp.float32),
        ],
    )
    return f(x1, y1, x2, y2, s, lab)


def kernel(boxes, scores, labels):
    pad = _PADN - _N
    x1 = jnp.pad(boxes[:, 0], (0, pad))
    y1 = jnp.pad(boxes[:, 1], (0, pad))
    x2 = jnp.pad(boxes[:, 2], (0, pad))
    y2 = jnp.pad(boxes[:, 3], (0, pad))
    s = jnp.pad(scores, (0, pad), constant_values=-1.0)
    labf = jnp.pad(labels.astype(jnp.float32), (0, pad), constant_values=-1.0)
    res = _sc_call(x1, y1, x2, y2, s, labf)
    return res.reshape(32, 16)[:30, :5]


# speedup vs baseline: 1.2557x; 1.2557x over previous
import jax
import jax.numpy as jnp
from jax import lax
from jax.experimental import pallas as pl
from jax.experimental.pallas import tpu as pltpu
from jax.experimental.pallas import tpu_sc as plsc

_N = 20000
_S = 16
_CH = 1280
_PADN = _S * _CH
_V = _CH // 16
_SCORE_THRESH = 0.2
_NMS_THRESH = 0.5
_BIG = float(2.0 ** 30)
_DNUMS = lax.GatherDimensionNumbers(
    offset_dims=(), collapsed_slice_dims=(0,), start_index_map=(0,))


def _permute(v, perm):
    return lax.gather(v, perm[:, None], _DNUMS, slice_sizes=(1,),
                      mode=lax.GatherScatterMode.PROMISE_IN_BOUNDS)


def _bfly_max(lane, best, idx):
    for d in (8, 4, 2, 1):
        perm = lane ^ d
        ob = _permute(best, perm)
        oi = _permute(idx, perm)
        better = ob > best
        tie = ob == best
        idx = jnp.where(better, oi,
                        jnp.where(tie, jnp.minimum(idx, oi), idx))
        best = jnp.maximum(best, ob)
    return best, idx


def _sc_body(x1h, y1h, x2h, y2h, sh, labh, outh,
             rx1, ry1, rx2, ry2, sv, lv,
             ox1, oy1, ox2, oy2, ar, esh, eso,
             stage, red, outv, shared):
    cid = lax.axis_index("c")
    sid = lax.axis_index("s")
    lane = lax.iota(jnp.int32, 16)

    @pl.when(cid == 0)
    def _main():
        base = sid * _CH
        pltpu.sync_copy(x1h.at[pl.ds(base, _CH)], rx1.at[pl.ds(0, _CH)])
        pltpu.sync_copy(y1h.at[pl.ds(base, _CH)], ry1.at[pl.ds(0, _CH)])
        pltpu.sync_copy(x2h.at[pl.ds(base, _CH)], rx2.at[pl.ds(0, _CH)])
        pltpu.sync_copy(y2h.at[pl.ds(base, _CH)], ry2.at[pl.ds(0, _CH)])
        pltpu.sync_copy(sh.at[pl.ds(base, _CH)], sv.at[pl.ds(0, _CH)])
        pltpu.sync_copy(labh.at[pl.ds(base, _CH)], lv.at[pl.ds(0, _CH)])

        def bmax_step(j, mv):
            sl = pl.ds(j * 16, 16)
            c = jnp.maximum(jnp.maximum(rx1[sl], ry1[sl]),
                            jnp.maximum(rx2[sl], ry2[sl]))
            c = jnp.where(sv[sl] >= _SCORE_THRESH, c, -jnp.inf)
            return jnp.maximum(mv, c)

        mv = lax.fori_loop(0, _V, bmax_step,
                           jnp.full((16,), -jnp.inf, jnp.float32),
                           unroll=4)
        lbv, _ = _bfly_max(lane, mv, lane.astype(jnp.float32))
        stage[...] = jnp.where(lane == 0, lbv, 0.0)
        pltpu.sync_copy(stage, shared.at[pl.ds(sid * 16, 16)])
        plsc.subcore_barrier()
        pltpu.sync_copy(shared.at[pl.ds(0, _S * 16)], red)

        def bred_step(j, m):
            rv = red[pl.ds(j * 16, 16)]
            return jnp.maximum(m, rv[0])

        bmax = lax.fori_loop(0, _S, bred_step, jnp.float32(-jnp.inf))
        plsc.subcore_barrier()

        def init_step(j, _):
            sl = pl.ds(j * 16, 16)
            labc = lv[sl]
            sc = sv[sl]
            off = labc * (bmax + 1.0)
            a1 = rx1[sl] + off
            b1 = ry1[sl] + off
            a2 = rx2[sl] + off
            b2 = ry2[sl] + off
            ox1[sl] = a1
            oy1[sl] = b1
            ox2[sl] = a2
            oy2[sl] = b2
            ar[sl] = (a2 - a1) * (b2 - b1)
            valid = sc >= _SCORE_THRESH
            esh[sl] = jnp.where(valid & (labc == 0.0), sc, -1.0)
            eso[sl] = jnp.where(valid & (labc != 0.0), sc, -1.0)
            return 0

        lax.fori_loop(0, _V, init_step, 0, unroll=4)

        def zero_step(j, _):
            outv[pl.ds(j * 16, 16)] = jnp.zeros((16,), jnp.float32)
            return 0

        lax.fori_loop(0, 32, zero_step, 0)

        def put_stage(buf, best, idx):
            mlocv, ilocv = _bfly_max(lane, best, idx)
            ilt = ilocv.astype(jnp.int32)
            li = ilt[0] - base
            valid_loc = (ilt[0] >= base) & (ilt[0] < base + _CH)
            li = jnp.where(valid_loc, li, 0)
            sl = pl.ds(li, 16)
            stage[...] = jnp.where(lane == 0, mlocv,
                         jnp.where(lane == 1, ilocv,
                         jnp.where(lane == 2, ox1[sl][0],
                         jnp.where(lane == 3, oy1[sl][0],
                         jnp.where(lane == 4, ox2[sl][0],
                         jnp.where(lane == 5, oy2[sl][0],
                         jnp.where(lane == 6, ar[sl][0],
                         jnp.where(lane == 7, rx1[sl][0],
                         jnp.where(lane == 8, ry1[sl][0],
                         jnp.where(lane == 9, rx2[sl][0],
                         jnp.where(lane == 10, ry2[sl][0],
                         jnp.where(lane == 11, sv[sl][0],
                                   0.0))))))))))))
            pltpu.sync_copy(
                stage, shared.at[pl.ds(buf * (_S * 16) + sid * 16, 16)])

        def plain_amax(es):
            def amax_step(j, bi):
                best, idx = bi
                v = es[pl.ds(j * 16, 16)]
                gt = v > best
                gidx = (base + j * 16 + lane).astype(jnp.float32)
                return (jnp.where(gt, v, best), jnp.where(gt, gidx, idx))

            return lax.fori_loop(
                0, _V, amax_step,
                (jnp.full((16,), -1.0, jnp.float32),
                 jnp.full((16,), _BIG, jnp.float32)), unroll=4)

        def phase(es, r0):
            best, idx = plain_amax(es)
            put_stage(0, best, idx)
            plsc.subcore_barrier()

            def body(p, r):
                cur = lax.rem(p, 2)
                nxt = lax.rem(p + 1, 2)
                pltpu.sync_copy(
                    shared.at[pl.ds(cur * (_S * 16), _S * 16)], red)

                def wred_step(j, wmi):
                    wm, wi = wmi
                    rv = red[pl.ds(j * 16, 16)]
                    mj = rv[0]
                    ij = rv[1]
                    better = mj > wm
                    tie = mj == wm
                    wi = jnp.where(better, ij,
                                   jnp.where(tie, jnp.minimum(wi, ij), wi))
                    return jnp.maximum(wm, mj), wi

                wm, wif = lax.fori_loop(0, _S, wred_step,
                                        (jnp.float32(-2.0),
                                         jnp.float32(_BIG)))
                wi = wif.astype(jnp.int32)
                ok = wm >= 0.0
                owner = lax.div(wi, jnp.int32(_CH))
                wrow = red[pl.ds(owner * 16, 16)]
                pox1 = jnp.where(ok, wrow[2], 1.0)
                poy1 = jnp.where(ok, wrow[3], 1.0)
                pox2 = jnp.where(ok, wrow[4], 0.0)
                poy2 = jnp.where(ok, wrow[5], 0.0)
                par = jnp.where(ok, wrow[6], 1.0)

                @pl.when(ok & (sid == 0))
                def _emit():
                    row = jnp.where(lane == 0, wrow[7],
                          jnp.where(lane == 1, wrow[8],
                          jnp.where(lane == 2, wrow[9],
                          jnp.where(lane == 3, wrow[10],
                          jnp.where(lane == 4, wrow[11],
                                    0.0)))))
                    outv[pl.ds(r * 16, 16)] = row

                def step(j, bi):
                    best, idx = bi
                    sl = pl.ds(j * 16, 16)
                    xx1 = jnp.maximum(pox1, ox1[sl])
                    yy1 = jnp.maximum(poy1, oy1[sl])
                    xx2 = jnp.minimum(pox2, ox2[sl])
                    yy2 = jnp.minimum(poy2, oy2[sl])
                    w = jnp.maximum(0.0, xx2 - xx1)
                    h = jnp.maximum(0.0, yy2 - yy1)
                    inter = w * h
                    iou = inter / (par + ar[sl] - inter + 1e-9)
                    v = jnp.where(iou > _NMS_THRESH, -1.0, es[sl])
                    es[sl] = v
                    gt = v > best
                    gidx = (base + j * 16 + lane).astype(jnp.float32)
                    return (jnp.where(gt, v, best),
                            jnp.where(gt, gidx, idx))

                best, idx = lax.fori_loop(
                    0, _V, step,
                    (jnp.full((16,), -1.0, jnp.float32),
                     jnp.full((16,), _BIG, jnp.float32)), unroll=4)
                put_stage(nxt, best, idx)
                plsc.subcore_barrier()
                return r + ok.astype(jnp.int32)

            return lax.fori_loop(0, 15, body, r0)

        r = phase(esh, jnp.int32(0))
        phase(eso, r)

        @pl.when(sid == 0)
        def _out():
            pltpu.sync_copy(outv, outh)


def _sc_call(x1, y1, x2, y2, s, lab):
    mesh = plsc.VectorSubcoreMesh(core_axis_name="c", subcore_axis_name="s")
    f = pl.kernel(
        _sc_body,
        out_type=jax.ShapeDtypeStruct((512,), jnp.float32),
        mesh=mesh,
        scratch_types=[
            pltpu.VMEM((_CH + 16,), jnp.float32) for _ in range(13)
        ] + [
            pltpu.VMEM((16,), jnp.float32),
            pltpu.VMEM((_S * 16,), jnp.float32),
            pltpu.VMEM((512,), jnp.float32),
            pltpu.VMEM_SHARED((2 * _S * 16,), jnp.float32),
        ],
    )
    return f(x1, y1, x2, y2, s, lab)


def kernel(boxes, scores, labels):
    pad = _PADN - _N
    x1 = jnp.pad(boxes[:, 0], (0, pad))
    y1 = jnp.pad(boxes[:, 1], (0, pad))
    x2 = jnp.pad(boxes[:, 2], (0, pad))
    y2 = jnp.pad(boxes[:, 3], (0, pad))
    s = jnp.pad(scores, (0, pad), constant_values=-1.0)
    labf = jnp.pad(labels.astype(jnp.float32), (0, pad), constant_values=-1.0)
    res = _sc_call(x1, y1, x2, y2, s, labf)
    return res.reshape(32, 16)[:30, :5]
